# Initial kernel scaffold; baseline (speedup 1.0000x reference)
import jax
import jax.numpy as jnp
from jax.experimental import pallas as pl


def kernel(m_ji, rbf_ji, atom_edge_index, W_rbf, W1, b1, W2, b2, W3, b3, W_out):
    raise NotImplementedError("write your pallas kernel here")


# speedup vs baseline: 2.4635x; 2.4635x over previous
import functools

import jax
import jax.numpy as jnp
from jax import lax
from jax.experimental import pallas as pl
from jax.experimental.pallas import tpu as pltpu
from jax.experimental.pallas import tpu_sc as plsc

N_NODES = 10000
N_EDGES = 320000
FEAT = 128
DIM_RBF = 16

NC = 2
NS = 16
NW = NC * NS
LANES = 128
ROWS = N_EDGES // LANES
N_PAD = 10240
ROWS_PER_TILE = N_PAD // NS

_BE = 4000


def _msg_body(m_ref, rbf_ref, wt_ref, out_ref):
    e = jnp.dot(rbf_ref[...], wt_ref[...], preferred_element_type=jnp.float32)
    out_ref[...] = e * m_ref[...]


def _scatter_body(msg_hbm, dst_hbm, zeros_hbm, out_hbm, msg_v, idx_v, acc_sh):
    cid = lax.axis_index("c")
    sid = lax.axis_index("s")

    pltpu.sync_copy(
        zeros_hbm.at[pl.ds(sid * ROWS_PER_TILE, ROWS_PER_TILE)],
        acc_sh.at[pl.ds(sid * ROWS_PER_TILE, ROWS_PER_TILE)],
    )
    plsc.subcore_barrier()

    w = sid * NC + cid
    rem = ROWS % NW
    base = w * (ROWS // NW) + jnp.minimum(w, rem)
    n = (ROWS // NW) + jnp.where(w < rem, 1, 0)

    def body(i, _):
        row = base + i
        pltpu.sync_copy(msg_hbm.at[pl.ds(row * LANES, LANES)], msg_v)
        pltpu.sync_copy(dst_hbm.at[row], idx_v)
        pltpu.sync_copy(msg_v, acc_sh.at[idx_v.at[0]], add=True)
        return _

    lax.fori_loop(0, n, body, None)
    plsc.subcore_barrier()

    pltpu.sync_copy(
        acc_sh.at[pl.ds(sid * ROWS_PER_TILE, ROWS_PER_TILE)],
        out_hbm.at[cid, pl.ds(sid * ROWS_PER_TILE, ROWS_PER_TILE)],
    )


def _mlp_body(a0_ref, a1_ref, w1_ref, b1_ref, w2_ref, b2_ref, w3_ref, b3_ref,
              wo_ref, out_ref):
    a = a0_ref[...] + a1_ref[...]
    a = jnp.dot(a, w1_ref[...], preferred_element_type=jnp.float32) + b1_ref[...]
    a = a * jax.nn.sigmoid(a)
    a = jnp.dot(a, w2_ref[...], preferred_element_type=jnp.float32) + b2_ref[...]
    a = a * jax.nn.sigmoid(a)
    a = jnp.dot(a, w3_ref[...], preferred_element_type=jnp.float32) + b3_ref[...]
    a = a * jax.nn.sigmoid(a)
    out_ref[...] = jnp.dot(a, wo_ref[...], preferred_element_type=jnp.float32)


def kernel(m_ji, rbf_ji, atom_edge_index, W_rbf, W1, b1, W2, b2, W3, b3, W_out):
    msg = pl.pallas_call(
        _msg_body,
        grid=(N_EDGES // _BE,),
        in_specs=[
            pl.BlockSpec((_BE, FEAT), lambda i: (i, 0)),
            pl.BlockSpec((_BE, DIM_RBF), lambda i: (i, 0)),
            pl.BlockSpec((DIM_RBF, FEAT), lambda i: (0, 0)),
        ],
        out_specs=pl.BlockSpec((_BE, FEAT), lambda i: (i, 0)),
        out_shape=jax.ShapeDtypeStruct((N_EDGES, FEAT), jnp.float32),
    )(m_ji, rbf_ji, W_rbf.T)

    dst3d = atom_edge_index[1].astype(jnp.int32).reshape(ROWS, 1, LANES)
    zeros = jnp.zeros((N_PAD, FEAT), jnp.float32)
    mesh = plsc.VectorSubcoreMesh(core_axis_name="c", subcore_axis_name="s")
    scatter = functools.partial(
        pl.kernel,
        out_type=jax.ShapeDtypeStruct((NC, N_PAD, FEAT), jnp.float32),
        mesh=mesh,
        scratch_types=[
            pltpu.VMEM((LANES, FEAT), jnp.float32),
            pltpu.VMEM((1, LANES), jnp.int32),
            pltpu.VMEM_SHARED((N_PAD, FEAT), jnp.float32),
        ],
    )(_scatter_body)
    acc2 = scatter(msg, dst3d, zeros)

    out = pl.pallas_call(
        _mlp_body,
        in_specs=[pl.BlockSpec(memory_space=pltpu.MemorySpace.VMEM)] * 9,
        out_specs=pl.BlockSpec(memory_space=pltpu.MemorySpace.VMEM),
        out_shape=jax.ShapeDtypeStruct((N_NODES, 1), jnp.float32),
    )(acc2[0, :N_NODES], acc2[1, :N_NODES], W1.T, b1.reshape(1, FEAT),
      W2.T, b2.reshape(1, FEAT), W3.T, b3.reshape(1, FEAT), W_out.T)
    return out


# speedup vs baseline: 4.6654x; 1.8938x over previous
import functools

import jax
import jax.numpy as jnp
from jax import lax
from jax.experimental import pallas as pl
from jax.experimental.pallas import tpu as pltpu
from jax.experimental.pallas import tpu_sc as plsc

N_NODES = 10000
N_EDGES = 320000
FEAT = 128
DIM_RBF = 16

NC = 2
NS = 16
NW = NC * NS
LANES = 128
ROWS = N_EDGES // LANES
N_PAD = 10240
ROWS_PER_TILE = N_PAD // NS

WROWS = ROWS // NW
REM_BASE = NW * WROWS

_BE = 6400


def _msg_body(m_ref, rbft_ref, wt_ref, out_ref):
    e = lax.dot_general(rbft_ref[...], wt_ref[...],
                        dimension_numbers=(((0,), (0,)), ((), ())),
                        preferred_element_type=jnp.float32)
    out_ref[...] = e * m_ref[...]


def _scatter_body(msg_hbm, dst_hbm, zeros_hbm, out_hbm,
                  msg_a, msg_b, idx_v, idx_l, acc_sh, sem_a, sem_b):
    cid = lax.axis_index("c")
    sid = lax.axis_index("s")
    w = sid * NC + cid
    base = w * WROWS

    pltpu.sync_copy(
        zeros_hbm.at[pl.ds(sid * ROWS_PER_TILE, ROWS_PER_TILE)],
        acc_sh.at[pl.ds(sid * ROWS_PER_TILE, ROWS_PER_TILE)],
    )
    pltpu.sync_copy(dst_hbm.at[pl.ds(base, WROWS)], idx_v)
    plsc.subcore_barrier()

    bufs = (msg_a, msg_b)
    sems = (sem_a, sem_b)

    def chunk(s):
        return msg_hbm.at[pl.ds((base + s) * LANES, LANES)]

    pltpu.async_copy(chunk(0), bufs[0], sems[0])
    pltpu.async_copy(chunk(1), bufs[1], sems[1])

    def step(g, _):
        for b in range(2):
            s = 2 * g + b
            pltpu.make_async_copy(chunk(s), bufs[b], sems[b]).wait()
            pltpu.sync_copy(bufs[b], acc_sh.at[idx_v.at[s, 0]], add=True)

            @pl.when(s + 2 < WROWS)
            def _():
                pltpu.async_copy(chunk(s + 2), bufs[b], sems[b])
        return _

    lax.fori_loop(0, WROWS // 2, step, None)

    @pl.when(w < ROWS - REM_BASE)
    def _():
        row = REM_BASE + w
        pltpu.sync_copy(msg_hbm.at[pl.ds(row * LANES, LANES)], msg_a)
        pltpu.sync_copy(dst_hbm.at[row], idx_l)
        pltpu.sync_copy(msg_a, acc_sh.at[idx_l.at[0]], add=True)

    plsc.subcore_barrier()

    pltpu.sync_copy(
        acc_sh.at[pl.ds(sid * ROWS_PER_TILE, ROWS_PER_TILE)],
        out_hbm.at[cid, pl.ds(sid * ROWS_PER_TILE, ROWS_PER_TILE)],
    )


def _mlp_body(acc_ref, w1_ref, b1_ref, w2_ref, b2_ref, w3_ref, b3_ref,
              wo_ref, out_ref):
    a = acc_ref[0] + acc_ref[1]
    a = jnp.dot(a, w1_ref[...], preferred_element_type=jnp.float32) + b1_ref[...]
    a = a * jax.nn.sigmoid(a)
    a = jnp.dot(a, w2_ref[...], preferred_element_type=jnp.float32) + b2_ref[...]
    a = a * jax.nn.sigmoid(a)
    a = jnp.dot(a, w3_ref[...], preferred_element_type=jnp.float32) + b3_ref[...]
    a = a * jax.nn.sigmoid(a)
    out_ref[...] = jnp.dot(a, wo_ref[...], preferred_element_type=jnp.float32)[:N_NODES]


def kernel(m_ji, rbf_ji, atom_edge_index, W_rbf, W1, b1, W2, b2, W3, b3, W_out):
    msg = pl.pallas_call(
        _msg_body,
        grid=(N_EDGES // _BE,),
        in_specs=[
            pl.BlockSpec((_BE, FEAT), lambda i: (i, 0)),
            pl.BlockSpec((DIM_RBF, _BE), lambda i: (0, i)),
            pl.BlockSpec((DIM_RBF, FEAT), lambda i: (0, 0)),
        ],
        out_specs=pl.BlockSpec((_BE, FEAT), lambda i: (i, 0)),
        out_shape=jax.ShapeDtypeStruct((N_EDGES, FEAT), jnp.float32),
    )(m_ji, rbf_ji.T, W_rbf.T)

    dst3d = atom_edge_index[1].astype(jnp.int32).reshape(ROWS, 1, LANES)
    zeros = jnp.zeros((N_PAD, FEAT), jnp.float32)
    mesh = plsc.VectorSubcoreMesh(core_axis_name="c", subcore_axis_name="s")
    scatter = functools.partial(
        pl.kernel,
        out_type=jax.ShapeDtypeStruct((NC, N_PAD, FEAT), jnp.float32),
        mesh=mesh,
        scratch_types=[
            pltpu.VMEM((LANES, FEAT), jnp.float32),
            pltpu.VMEM((LANES, FEAT), jnp.float32),
            pltpu.VMEM((WROWS, 1, LANES), jnp.int32),
            pltpu.VMEM((1, LANES), jnp.int32),
            pltpu.VMEM_SHARED((N_PAD, FEAT), jnp.float32),
            pltpu.SemaphoreType.DMA,
            pltpu.SemaphoreType.DMA,
        ],
    )(_scatter_body)
    acc2 = scatter(msg, dst3d, zeros)

    out = pl.pallas_call(
        _mlp_body,
        in_specs=[pl.BlockSpec(memory_space=pltpu.MemorySpace.VMEM)] * 8,
        out_specs=pl.BlockSpec(memory_space=pltpu.MemorySpace.VMEM),
        out_shape=jax.ShapeDtypeStruct((N_NODES, 1), jnp.float32),
    )(acc2, W1.T, b1.reshape(1, FEAT), W2.T, b2.reshape(1, FEAT),
      W3.T, b3.reshape(1, FEAT), W_out.T)
    return out
